# baseline (speedup 1.0000x reference)
import jax
import jax.numpy as jnp
from jax.experimental import pallas as pl

L = 16
D = H = W = 16
N = D * H * W
ALPHA = 80.0
BETA = 0.5
GAMMA = 3.0
RANK = 16

_A = -1.0 / (2.0 * ALPHA * ALPHA)
_B = -1.0 / (2.0 * BETA * BETA)
_C = -1.0 / (2.0 * GAMMA * GAMMA)


def _iota(shape, dim):
    return jax.lax.broadcasted_iota(jnp.int32, shape, dim)


def _kron_pair(coeff):
    r = _iota((256, 256), 0)
    c = _iota((256, 256), 1)
    dq = ((r >> 4) - (c >> 4)).astype(jnp.float32)
    ds = ((r & 15) - (c & 15)).astype(jnp.float32)
    return jnp.exp(coeff * (dq * dq + ds * ds))


def _line(coeff):
    d = (_iota((16, 16), 0) - _iota((16, 16), 1)).astype(jnp.float32)
    return jnp.exp(coeff * d * d)


def _dot(a, b):
    return jax.lax.dot_general(a, b, (((1,), (0,)), ((), ())),
                               preferred_element_type=jnp.float32)


def _eye16():
    return (_iota((16, 16), 0) == _iota((16, 16), 1)).astype(jnp.float32)


def _crf_kernel(u_ref, f_ref, fz_ref, sw_ref, bw_ref, cm_ref, out_ref):
    u = u_ref[...]
    fr = f_ref[...]

    mx = jnp.max(u, axis=0, keepdims=True)
    eu = jnp.exp(u - mx)
    qs = eu / jnp.sum(eu, axis=0, keepdims=True)

    gqs_a = _kron_pair(_A)
    gqs_c = _kron_pair(_C)
    gp_a = _line(_A)
    gp_c = _line(_C)
    eye = _eye16()

    tf = 2.0 * fr
    base = jnp.exp(_B * fr * fr)
    phi_rows = [base]
    for k in range(1, RANK):
        phi_rows.append(phi_rows[-1] * tf * (1.0 / (k ** 0.5)))

    v0 = jnp.concatenate([qs * phi_rows[k] for k in range(RANK)],
                         axis=0)
    t1 = _dot(v0.reshape(N, 256).astype(jnp.bfloat16),
              gqs_a.astype(jnp.bfloat16))
    t2 = jnp.transpose(t1.reshape(256, N))
    t3 = _dot(gp_a, t2.reshape(16, 16 * N))
    t4 = t3.reshape(N, 256)
    v0n = jnp.concatenate(phi_rows, axis=0)
    phi_t = jax.lax.dot_general(v0n, eye, (((0,), (0,)), ((), ())),
                                preferred_element_type=jnp.float32)
    expand = (_iota((16, 256), 0)
              == _iota((16, 256), 1) // 16).astype(jnp.float32)
    phi_exp = _dot(phi_t, expand)
    sel_l = (_iota((256, 16), 0) % 16
             == _iota((256, 16), 1)).astype(jnp.float32)
    yb_vox = _dot(t4 * phi_exp, sel_l)
    yb = jax.lax.dot_general(eye, yb_vox, (((1,), (1,)), ((), ())),
                             preferred_element_type=jnp.float32)

    n1 = _dot(v0n.reshape(256, 256), gqs_a)
    kron_igz_a = ((_iota((256, 256), 0) >> 4 == _iota((256, 256), 1) >> 4)
                  .astype(jnp.float32)
                  * jnp.exp(_A * ((_iota((256, 256), 0) & 15)
                                  - (_iota((256, 256), 1) & 15)).astype(
                      jnp.float32) ** 2))
    n2 = _dot(kron_igz_a, n1)

    fzv = fz_ref[...]
    base_z = jnp.exp(_B * fzv * fzv)
    tfz = 2.0 * fzv
    phi_slabs = [base_z]
    for k in range(1, RANK):
        phi_slabs.append(phi_slabs[-1] * tfz * (1.0 / (k ** 0.5)))
    phi_cat = jnp.concatenate(phi_slabs, axis=0)

    sum_kz = ((_iota((16, 256), 1) & 15)
              == _iota((16, 256), 0)).astype(jnp.float32)
    nb_z = _dot(sum_kz, n2 * phi_cat)
    nb = jnp.concatenate(
        [jnp.broadcast_to(nb_z[z:z + 1, :], (16, 256)) for z in range(16)],
        axis=1)

    s1 = _dot(qs.reshape(256, 256), gqs_c)
    s2 = jnp.transpose(s1)
    kron_igz = ((_iota((256, 256), 0) >> 4 == _iota((256, 256), 1) >> 4)
                .astype(jnp.float32)
                * jnp.exp(_C * ((_iota((256, 256), 0) & 15)
                                - (_iota((256, 256), 1) & 15)).astype(
                    jnp.float32) ** 2))
    s3 = _dot(s2, kron_igz)
    ys = jnp.transpose(s3).reshape(16, N)

    lane = _iota((1, N), 1)
    zc = (lane >> 8).astype(jnp.float32)
    yc = ((lane >> 4) & 15).astype(jnp.float32)
    xc = (lane & 15).astype(jnp.float32)
    gz = jnp.zeros((1, N), jnp.float32)
    gy = jnp.zeros((1, N), jnp.float32)
    gx = jnp.zeros((1, N), jnp.float32)
    for j in range(16):
        gz = gz + jnp.exp(_C * (zc - j) * (zc - j))
        gy = gy + jnp.exp(_C * (yc - j) * (yc - j))
        gx = gx + jnp.exp(_C * (xc - j) * (xc - j))
    ns = gz * gy * gx

    cb = jnp.dot(cm_ref[...], bw_ref[...], preferred_element_type=jnp.float32)
    cs = jnp.dot(cm_ref[...], sw_ref[...], preferred_element_type=jnp.float32)
    m = (jnp.dot(cs, ys, preferred_element_type=jnp.float32) / ns
         + jnp.dot(cb, yb, preferred_element_type=jnp.float32) / nb)
    q = u + m
    qmx = jnp.max(q, axis=0, keepdims=True)
    eq = jnp.exp(q - qmx)
    out_ref[...] = eq / jnp.sum(eq, axis=0, keepdims=True)


@jax.jit
def kernel(U, I, spatial_ker_weights, bilateral_ker_weights,
           compatibility_matrix):
    u_flat = U[0].reshape(L, N)
    feat = I.reshape(1, N)
    feat_z = I.reshape(16, 256)
    out = pl.pallas_call(
        _crf_kernel,
        grid=(1,),
        in_specs=[
            pl.BlockSpec((L, N), lambda j: (0, 0)),
            pl.BlockSpec((1, N), lambda j: (0, 0)),
            pl.BlockSpec((16, 256), lambda j: (0, 0)),
            pl.BlockSpec((L, L), lambda j: (0, 0)),
            pl.BlockSpec((L, L), lambda j: (0, 0)),
            pl.BlockSpec((L, L), lambda j: (0, 0)),
        ],
        out_specs=pl.BlockSpec((L, N), lambda j: (0, 0)),
        out_shape=jax.ShapeDtypeStruct((L, N), jnp.float32),
    )(u_flat, feat, feat_z, spatial_ker_weights, bilateral_ker_weights,
      compatibility_matrix)
    return out.reshape(1, L, D, H, W)


# speedup vs baseline: 1.0103x; 1.0103x over previous
import jax
import jax.numpy as jnp
from jax.experimental import pallas as pl

L = 16
D = H = W = 16
N = D * H * W
ALPHA = 80.0
BETA = 0.5
GAMMA = 3.0
RANK = 16

_A = -1.0 / (2.0 * ALPHA * ALPHA)
_B = -1.0 / (2.0 * BETA * BETA)
_C = -1.0 / (2.0 * GAMMA * GAMMA)


def _iota(shape, dim):
    return jax.lax.broadcasted_iota(jnp.int32, shape, dim)


def _kron_pair(coeff):
    r = _iota((256, 256), 0)
    c = _iota((256, 256), 1)
    dq = ((r >> 4) - (c >> 4)).astype(jnp.float32)
    ds = ((r & 15) - (c & 15)).astype(jnp.float32)
    return jnp.exp(coeff * (dq * dq + ds * ds))


def _line(coeff):
    d = (_iota((16, 16), 0) - _iota((16, 16), 1)).astype(jnp.float32)
    return jnp.exp(coeff * d * d)


def _dot(a, b):
    return jax.lax.dot_general(a, b, (((1,), (0,)), ((), ())),
                               preferred_element_type=jnp.float32)


def _eye16():
    return (_iota((16, 16), 0) == _iota((16, 16), 1)).astype(jnp.float32)


def _crf_kernel(u_ref, f_ref, fz_ref, sw_ref, bw_ref, cm_ref, out_ref):
    u = u_ref[...]
    fr = f_ref[...]

    mx = jnp.max(u, axis=0, keepdims=True)
    eu = jnp.exp(u - mx)
    qs = eu / jnp.sum(eu, axis=0, keepdims=True)

    gqs_a = _kron_pair(_A)
    gqs_c = _kron_pair(_C)
    gp_a = _line(_A)
    gp_c = _line(_C)
    eye = _eye16()

    tf = 2.0 * fr
    base = jnp.exp(_B * fr * fr)
    phi_rows = [base]
    for k in range(1, RANK):
        phi_rows.append(phi_rows[-1] * tf * (1.0 / (k ** 0.5)))

    v0 = jnp.concatenate([qs * phi_rows[k] for k in range(RANK)],
                         axis=0)
    t1 = _dot(v0.reshape(N, 256), gqs_a)
    t2 = jnp.transpose(t1.reshape(256, N))
    t3 = _dot(gp_a, t2.reshape(16, 16 * N))
    t4 = t3.reshape(N, 256)
    v0n = jnp.concatenate(phi_rows, axis=0)
    phi_t = jax.lax.dot_general(v0n, eye, (((0,), (0,)), ((), ())),
                                preferred_element_type=jnp.float32)
    expand = (_iota((16, 256), 0)
              == _iota((16, 256), 1) // 16).astype(jnp.float32)
    phi_exp = _dot(phi_t, expand)
    sel_l = (_iota((256, 16), 0) % 16
             == _iota((256, 16), 1)).astype(jnp.float32)
    yb_vox = _dot(t4 * phi_exp, sel_l)
    yb = jax.lax.dot_general(eye, yb_vox, (((1,), (1,)), ((), ())),
                             preferred_element_type=jnp.float32)

    n1 = _dot(v0n.reshape(256, 256), gqs_a)
    kron_igz_a = ((_iota((256, 256), 0) >> 4 == _iota((256, 256), 1) >> 4)
                  .astype(jnp.float32)
                  * jnp.exp(_A * ((_iota((256, 256), 0) & 15)
                                  - (_iota((256, 256), 1) & 15)).astype(
                      jnp.float32) ** 2))
    n2 = _dot(kron_igz_a, n1)

    fzv = fz_ref[...]
    base_z = jnp.exp(_B * fzv * fzv)
    tfz = 2.0 * fzv
    phi_slabs = [base_z]
    for k in range(1, RANK):
        phi_slabs.append(phi_slabs[-1] * tfz * (1.0 / (k ** 0.5)))
    phi_cat = jnp.concatenate(phi_slabs, axis=0)

    sum_kz = ((_iota((16, 256), 1) & 15)
              == _iota((16, 256), 0)).astype(jnp.float32)
    nb_z = _dot(sum_kz, n2 * phi_cat)
    nb = jnp.concatenate(
        [jnp.broadcast_to(nb_z[z:z + 1, :], (16, 256)) for z in range(16)],
        axis=1)

    s1 = _dot(qs.reshape(256, 256), gqs_c)
    s2 = jnp.transpose(s1)
    kron_igz = ((_iota((256, 256), 0) >> 4 == _iota((256, 256), 1) >> 4)
                .astype(jnp.float32)
                * jnp.exp(_C * ((_iota((256, 256), 0) & 15)
                                - (_iota((256, 256), 1) & 15)).astype(
                    jnp.float32) ** 2))
    s3 = _dot(s2, kron_igz)
    ys = jnp.transpose(s3).reshape(16, N)

    lane = _iota((1, N), 1)
    zc = (lane >> 8).astype(jnp.float32)
    yc = ((lane >> 4) & 15).astype(jnp.float32)
    xc = (lane & 15).astype(jnp.float32)
    gz = jnp.zeros((1, N), jnp.float32)
    gy = jnp.zeros((1, N), jnp.float32)
    gx = jnp.zeros((1, N), jnp.float32)
    for j in range(16):
        gz = gz + jnp.exp(_C * (zc - j) * (zc - j))
        gy = gy + jnp.exp(_C * (yc - j) * (yc - j))
        gx = gx + jnp.exp(_C * (xc - j) * (xc - j))
    ns = gz * gy * gx

    cb = jnp.dot(cm_ref[...], bw_ref[...], preferred_element_type=jnp.float32)
    cs = jnp.dot(cm_ref[...], sw_ref[...], preferred_element_type=jnp.float32)
    m = (jnp.dot(cs, ys, preferred_element_type=jnp.float32) / ns
         + jnp.dot(cb, yb, preferred_element_type=jnp.float32) / nb)
    q = u + m
    qmx = jnp.max(q, axis=0, keepdims=True)
    eq = jnp.exp(q - qmx)
    out_ref[...] = eq / jnp.sum(eq, axis=0, keepdims=True)


@jax.jit
def kernel(U, I, spatial_ker_weights, bilateral_ker_weights,
           compatibility_matrix):
    u_flat = U[0].reshape(L, N)
    feat = I.reshape(1, N)
    feat_z = I.reshape(16, 256)
    out = pl.pallas_call(
        _crf_kernel,
        grid=(1,),
        in_specs=[
            pl.BlockSpec((L, N), lambda j: (0, 0)),
            pl.BlockSpec((1, N), lambda j: (0, 0)),
            pl.BlockSpec((16, 256), lambda j: (0, 0)),
            pl.BlockSpec((L, L), lambda j: (0, 0)),
            pl.BlockSpec((L, L), lambda j: (0, 0)),
            pl.BlockSpec((L, L), lambda j: (0, 0)),
        ],
        out_specs=pl.BlockSpec((L, N), lambda j: (0, 0)),
        out_shape=jax.ShapeDtypeStruct((L, N), jnp.float32),
    )(u_flat, feat, feat_z, spatial_ker_weights, bilateral_ker_weights,
      compatibility_matrix)
    return out.reshape(1, L, D, H, W)
